# initial kernel scaffold (unmeasured)
import jax
import jax.numpy as jnp
from jax import lax
from jax.experimental import pallas as pl
from jax.experimental.pallas import tpu as pltpu

N_DEV = 16

RING = [0, 4, 8, 12, 13, 9, 5, 1, 2, 6, 10, 14, 15, 11, 7, 3]
RPOS = [0] * N_DEV
NEXT_ID = [0] * N_DEV
PREV_ID = [0] * N_DEV
for _j, _p in enumerate(RING):
    RPOS[_p] = _j
    NEXT_ID[_p] = RING[(_j + 1) % N_DEV]
    PREV_ID[_p] = RING[(_j - 1) % N_DEV]

_DEV_ID_TYPE = getattr(pl, "DeviceIdType", None) or pltpu.DeviceIdType


def _lut(idx, table):
    out = jnp.int32(table[0])
    for k in range(1, len(table)):
        out = jnp.where(idx == k, jnp.int32(table[k]), out)
    return out


def kernel(x, w_mat):
    m, k_sh = x.shape
    _, n = w_mat.shape
    ch = m // N_DEV

    def body(x_ref, w_ref, out_ref, comm_ref, stage_ref,
             send_sems, recv_sems, out_sem, credit_sem):
        my = lax.axis_index("i")
        r = _lut(my, RPOS)
        right = _lut(my, NEXT_ID)
        left = _lut(my, PREV_ID)

        barrier_sem = pltpu.get_barrier_semaphore()
        for nbr in (left, right):
            pl.semaphore_signal(barrier_sem, inc=1, device_id=(nbr,),
                                device_id_type=_DEV_ID_TYPE.MESH)
        pl.semaphore_wait(barrier_sem, 2)

        def partial_f32(c):
            xc = x_ref[pl.ds(c * ch, ch), :]
            return lax.dot_general(
                xc, w_ref[:, :], (((1,), (0,)), ((), ())),
                preferred_element_type=jnp.float32)

        def gelu(y):
            k0 = 0.7978845608028654
            return 0.5 * y * (1.0 + jnp.tanh(k0 * (y + 0.044715 * y * y * y)))

        def store_chunk(c, vals_f32):
            stage_ref[...] = gelu(vals_f32)
            cp = pltpu.make_async_copy(
                stage_ref, out_ref.at[pl.ds(c * ch, ch), :], out_sem)
            cp.start()
            cp.wait()

        comm_ref[0, :, :] = partial_f32(r).astype(jnp.bfloat16)

        n_hops = 2 * (N_DEV - 1)
        for h in range(n_hops):
            s = h % 2
            d = (h + 1) % 2
            if h > 0:
                pl.semaphore_wait(credit_sem, 1)
            rdma = pltpu.make_async_remote_copy(
                src_ref=comm_ref.at[s],
                dst_ref=comm_ref.at[d],
                send_sem=send_sems.at[s],
                recv_sem=recv_sems.at[d],
                device_id=(right,),
                device_id_type=_DEV_ID_TYPE.MESH,
            )
            rdma.start()
            rdma.wait()
            if h < n_hops - 1:
                pl.semaphore_signal(credit_sem, inc=1, device_id=(left,),
                                    device_id_type=_DEV_ID_TYPE.MESH)
            if h < N_DEV - 1:
                c = lax.rem(r - h - 1 + N_DEV, N_DEV)
                acc = comm_ref[d].astype(jnp.float32) + partial_f32(c)
                comm_ref[d, :, :] = acc.astype(jnp.bfloat16)
                if h == N_DEV - 2:
                    store_chunk(lax.rem(r + 1, N_DEV), acc)
            else:
                g = h - (N_DEV - 1)
                c = lax.rem(r - g + N_DEV, N_DEV)
                store_chunk(c, comm_ref[d].astype(jnp.float32))

    return pl.pallas_call(
        body,
        out_shape=jax.ShapeDtypeStruct((m, n), jnp.float32),
        in_specs=[
            pl.BlockSpec(memory_space=pltpu.VMEM),
            pl.BlockSpec(memory_space=pltpu.VMEM),
        ],
        out_specs=pl.BlockSpec(memory_space=pltpu.ANY),
        scratch_shapes=[
            pltpu.VMEM((2, ch, n), jnp.bfloat16),
            pltpu.VMEM((ch, n), jnp.float32),
            pltpu.SemaphoreType.DMA((2,)),
            pltpu.SemaphoreType.DMA((2,)),
            pltpu.SemaphoreType.DMA,
            pltpu.SemaphoreType.REGULAR,
        ],
        compiler_params=pltpu.CompilerParams(collective_id=0),
    )(x, w_mat)


# baseline (device time: 1646216 ns/iter reference)
import jax
import jax.numpy as jnp
from jax import lax
from jax.experimental import pallas as pl
from jax.experimental.pallas import tpu as pltpu

N_DEV = 16

RING = [0, 4, 8, 12, 13, 9, 5, 1, 2, 6, 10, 14, 15, 11, 7, 3]
RPOS = [0] * N_DEV
NEXT_ID = [0] * N_DEV
PREV_ID = [0] * N_DEV
for _j, _p in enumerate(RING):
    RPOS[_p] = _j
    NEXT_ID[_p] = RING[(_j + 1) % N_DEV]
    PREV_ID[_p] = RING[(_j - 1) % N_DEV]

_DEV_ID_TYPE = getattr(pl, "DeviceIdType", None) or pltpu.DeviceIdType


def _lut(idx, table):
    out = jnp.int32(table[0])
    for k in range(1, len(table)):
        out = jnp.where(idx == k, jnp.int32(table[k]), out)
    return out


def kernel(x, w_mat):
    x = x.astype(jnp.bfloat16)
    w_mat = w_mat.astype(jnp.bfloat16)
    m, k_sh = x.shape
    _, n = w_mat.shape
    ch = m // N_DEV

    def body(x_ref, w_ref, out_ref, comm_ref, stage_ref,
             send_sems, recv_sems, out_sem, credit_sem):
        my = lax.axis_index("i")
        r = _lut(my, RPOS)
        right = _lut(my, NEXT_ID)
        left = _lut(my, PREV_ID)

        barrier_sem = pltpu.get_barrier_semaphore()
        for nbr in (left, right):
            pl.semaphore_signal(barrier_sem, inc=1, device_id=(nbr,),
                                device_id_type=_DEV_ID_TYPE.MESH)
        pl.semaphore_wait(barrier_sem, 2)

        def credit_left():
            pl.semaphore_signal(credit_sem, inc=1, device_id=(left,),
                                device_id_type=_DEV_ID_TYPE.MESH)

        def partial_f32(c):
            xc = x_ref[pl.ds(c * ch, ch), :]
            return lax.dot_general(
                xc, w_ref[:, :], (((1,), (0,)), ((), ())),
                preferred_element_type=jnp.float32)

        def gelu(y):
            k0 = 0.7978845608028654
            return 0.5 * y * (1.0 + jnp.tanh(k0 * (y + 0.044715 * y * y * y)))

        def store_chunk(c, vals_f32):
            stage_ref[...] = gelu(vals_f32)
            cp = pltpu.make_async_copy(
                stage_ref, out_ref.at[pl.ds(c * ch, ch), :], out_sem)
            cp.start()
            cp.wait()

        def hop(s, d):
            pl.semaphore_wait(credit_sem, 1)
            rdma = pltpu.make_async_remote_copy(
                src_ref=comm_ref.at[s],
                dst_ref=comm_ref.at[d],
                send_sem=send_sems.at[s],
                recv_sem=recv_sems.at[d],
                device_id=(right,),
                device_id_type=_DEV_ID_TYPE.MESH,
            )
            rdma.start()
            rdma.wait()
            credit_left()

        def rs_hop(h, s, d):
            hop(s, d)
            c = lax.rem(r - h - 1 + N_DEV, N_DEV)
            acc = comm_ref[d].astype(jnp.float32) + partial_f32(c)
            comm_ref[d, :, :] = acc.astype(jnp.bfloat16)

        def ag_hop(g, s, d):
            hop(s, d)
            store_chunk(lax.rem(r - g + N_DEV, N_DEV),
                        comm_ref[d].astype(jnp.float32))

        comm_ref[0, :, :] = partial_f32(r).astype(jnp.bfloat16)

        credit_left()

        def rs_pair(i, carry):
            rs_hop(2 * i, 0, 1)
            rs_hop(2 * i + 1, 1, 0)
            return carry
        lax.fori_loop(0, (N_DEV - 1) // 2, rs_pair, 0)
        rs_hop(N_DEV - 2, 0, 1)

        store_chunk(lax.rem(r + 1, N_DEV), comm_ref[1].astype(jnp.float32))

        def ag_pair(j, carry):
            ag_hop(2 * j, 1, 0)
            ag_hop(2 * j + 1, 0, 1)
            return carry
        lax.fori_loop(0, (N_DEV - 1) // 2, ag_pair, 0)
        ag_hop(N_DEV - 2, 1, 0)

        pl.semaphore_wait(credit_sem, 1)

    return pl.pallas_call(
        body,
        out_shape=jax.ShapeDtypeStruct((m, n), jnp.float32),
        in_specs=[
            pl.BlockSpec(memory_space=pltpu.VMEM),
            pl.BlockSpec(memory_space=pltpu.VMEM),
        ],
        out_specs=pl.BlockSpec(memory_space=pl.ANY),
        scratch_shapes=[
            pltpu.VMEM((2, ch, n), jnp.bfloat16),
            pltpu.VMEM((ch, n), jnp.float32),
            pltpu.SemaphoreType.DMA((2,)),
            pltpu.SemaphoreType.DMA((2,)),
            pltpu.SemaphoreType.DMA,
            pltpu.SemaphoreType.REGULAR,
        ],
        compiler_params=pltpu.CompilerParams(
            collective_id=0, vmem_limit_bytes=100 * 1024 * 1024),
    )(x, w_mat)


# device time: 960269 ns/iter; 1.7143x vs baseline; 1.7143x over previous
import jax
import jax.numpy as jnp
from jax import lax
from jax.experimental import pallas as pl
from jax.experimental.pallas import tpu as pltpu

N_DEV = 16

RING = [0, 4, 8, 12, 13, 9, 5, 1, 2, 6, 10, 14, 15, 11, 7, 3]
RPOS = [0] * N_DEV
NEXT_ID = [0] * N_DEV
PREV_ID = [0] * N_DEV
for _j, _p in enumerate(RING):
    RPOS[_p] = _j
    NEXT_ID[_p] = RING[(_j + 1) % N_DEV]
    PREV_ID[_p] = RING[(_j - 1) % N_DEV]

_DEV_ID_TYPE = getattr(pl, "DeviceIdType", None) or pltpu.DeviceIdType


def _lut(idx, table):
    out = jnp.int32(table[0])
    for k in range(1, len(table)):
        out = jnp.where(idx == k, jnp.int32(table[k]), out)
    return out


def kernel(x, w_mat):
    x = x.astype(jnp.bfloat16)
    w_mat = w_mat.astype(jnp.bfloat16)
    m, k_sh = x.shape
    _, n = w_mat.shape
    ch = m // N_DEV
    n2 = n // 2

    def body(x_ref, w_ref, out_ref, comm_r, comm_l, stage_r, stage_l,
             send_sems_r, recv_sems_r, send_sems_l, recv_sems_l,
             out_sem_r, out_sem_l, credit_r, credit_l):
        my = lax.axis_index("i")
        r = _lut(my, RPOS)
        right = _lut(my, NEXT_ID)
        left = _lut(my, PREV_ID)

        barrier_sem = pltpu.get_barrier_semaphore()
        for nbr in (left, right):
            pl.semaphore_signal(barrier_sem, inc=1, device_id=(nbr,),
                                device_id_type=_DEV_ID_TYPE.MESH)
        pl.semaphore_wait(barrier_sem, 2)

        def signal(sem, dev):
            pl.semaphore_signal(sem, inc=1, device_id=(dev,),
                                device_id_type=_DEV_ID_TYPE.MESH)

        def partial_half(c, half):
            xc = x_ref[pl.ds(c * ch, ch), :]
            wc = w_ref[:, half * n2:(half + 1) * n2]
            return lax.dot_general(
                xc, wc, (((1,), (0,)), ((), ())),
                preferred_element_type=jnp.float32)

        def gelu(y):
            k0 = 0.7978845608028654
            return 0.5 * y * (1.0 + jnp.tanh(k0 * (y + 0.044715 * y * y * y)))

        def store_half(c, half, stage, sem, vals_f32):
            stage[...] = gelu(vals_f32)
            cp = pltpu.make_async_copy(
                stage,
                out_ref.at[pl.ds(c * ch, ch), half * n2:(half + 1) * n2],
                sem)
            cp.start()
            cp.wait()

        def start_hop(s, d):
            pl.semaphore_wait(credit_r, 1)
            pl.semaphore_wait(credit_l, 1)
            rdma_r = pltpu.make_async_remote_copy(
                src_ref=comm_r.at[s], dst_ref=comm_r.at[d],
                send_sem=send_sems_r.at[s], recv_sem=recv_sems_r.at[d],
                device_id=(right,), device_id_type=_DEV_ID_TYPE.MESH)
            rdma_l = pltpu.make_async_remote_copy(
                src_ref=comm_l.at[s], dst_ref=comm_l.at[d],
                send_sem=send_sems_l.at[s], recv_sem=recv_sems_l.at[d],
                device_id=(left,), device_id_type=_DEV_ID_TYPE.MESH)
            rdma_r.start()
            rdma_l.start()
            return rdma_r, rdma_l

        def finish_hop(rdma_r, rdma_l):
            rdma_r.wait()
            rdma_l.wait()
            signal(credit_r, left)
            signal(credit_l, right)

        def rs_hop(h, s, d):
            rdma_r, rdma_l = start_hop(s, d)
            p_r = partial_half(lax.rem(r - h - 1 + N_DEV, N_DEV), 0)
            p_l = partial_half(lax.rem(r + h + 1, N_DEV), 1)
            finish_hop(rdma_r, rdma_l)
            comm_r[d, :, :] = (comm_r[d].astype(jnp.float32)
                               + p_r).astype(jnp.bfloat16)
            comm_l[d, :, :] = (comm_l[d].astype(jnp.float32)
                               + p_l).astype(jnp.bfloat16)

        def ag_hop(g, s, d):
            rdma_r, rdma_l = start_hop(s, d)
            finish_hop(rdma_r, rdma_l)
            store_half(lax.rem(r - g + N_DEV, N_DEV), 0, stage_r, out_sem_r,
                       comm_r[d].astype(jnp.float32))
            store_half(lax.rem(r + g, N_DEV), 1, stage_l, out_sem_l,
                       comm_l[d].astype(jnp.float32))

        comm_r[0, :, :] = partial_half(r, 0).astype(jnp.bfloat16)
        comm_l[0, :, :] = partial_half(r, 1).astype(jnp.bfloat16)

        signal(credit_r, left)
        signal(credit_l, right)

        def rs_pair(i, carry):
            rs_hop(2 * i, 0, 1)
            rs_hop(2 * i + 1, 1, 0)
            return carry
        lax.fori_loop(0, (N_DEV - 1) // 2, rs_pair, 0)
        rs_hop(N_DEV - 2, 0, 1)

        store_half(lax.rem(r + 1, N_DEV), 0, stage_r, out_sem_r,
                   comm_r[1].astype(jnp.float32))
        store_half(lax.rem(r - 1 + N_DEV, N_DEV), 1, stage_l, out_sem_l,
                   comm_l[1].astype(jnp.float32))

        def ag_pair(j, carry):
            ag_hop(2 * j, 1, 0)
            ag_hop(2 * j + 1, 0, 1)
            return carry
        lax.fori_loop(0, (N_DEV - 1) // 2, ag_pair, 0)
        ag_hop(N_DEV - 2, 1, 0)

        pl.semaphore_wait(credit_r, 1)
        pl.semaphore_wait(credit_l, 1)

    return pl.pallas_call(
        body,
        out_shape=jax.ShapeDtypeStruct((m, n), jnp.float32),
        in_specs=[
            pl.BlockSpec(memory_space=pltpu.VMEM),
            pl.BlockSpec(memory_space=pltpu.VMEM),
        ],
        out_specs=pl.BlockSpec(memory_space=pl.ANY),
        scratch_shapes=[
            pltpu.VMEM((2, ch, n2), jnp.bfloat16),
            pltpu.VMEM((2, ch, n2), jnp.bfloat16),
            pltpu.VMEM((ch, n2), jnp.float32),
            pltpu.VMEM((ch, n2), jnp.float32),
            pltpu.SemaphoreType.DMA((2,)),
            pltpu.SemaphoreType.DMA((2,)),
            pltpu.SemaphoreType.DMA((2,)),
            pltpu.SemaphoreType.DMA((2,)),
            pltpu.SemaphoreType.DMA,
            pltpu.SemaphoreType.DMA,
            pltpu.SemaphoreType.REGULAR,
            pltpu.SemaphoreType.REGULAR,
        ],
        compiler_params=pltpu.CompilerParams(
            collective_id=0, vmem_limit_bytes=100 * 1024 * 1024),
    )(x, w_mat)


# device time: 867726 ns/iter; 1.8972x vs baseline; 1.1067x over previous
import jax
import jax.numpy as jnp
from jax import lax
from jax.experimental import pallas as pl
from jax.experimental.pallas import tpu as pltpu

N_DEV = 16

RING = [0, 4, 8, 12, 13, 9, 5, 1, 2, 6, 10, 14, 15, 11, 7, 3]
RPOS = [0] * N_DEV
NEXT_ID = [0] * N_DEV
PREV_ID = [0] * N_DEV
for _j, _p in enumerate(RING):
    RPOS[_p] = _j
    NEXT_ID[_p] = RING[(_j + 1) % N_DEV]
    PREV_ID[_p] = RING[(_j - 1) % N_DEV]

_DEV_ID_TYPE = getattr(pl, "DeviceIdType", None) or pltpu.DeviceIdType


def _lut(idx, table):
    out = jnp.int32(table[0])
    for k in range(1, len(table)):
        out = jnp.where(idx == k, jnp.int32(table[k]), out)
    return out


def kernel(x, w_mat):
    x = x.astype(jnp.bfloat16)
    w_mat = w_mat.astype(jnp.bfloat16)
    m, k_sh = x.shape
    _, n = w_mat.shape
    ch = m // N_DEV
    n2 = n // 2

    def body(x_ref, w_ref, out_ref, comm_r, comm_l, stage_r, stage_l,
             send_sems_r, recv_sems_r, send_sems_l, recv_sems_l,
             out_sem_r, out_sem_l, credit_r, credit_l):
        my = lax.axis_index("i")
        r = _lut(my, RPOS)
        right = _lut(my, NEXT_ID)
        left = _lut(my, PREV_ID)

        barrier_sem = pltpu.get_barrier_semaphore()
        for nbr in (left, right):
            pl.semaphore_signal(barrier_sem, inc=1, device_id=(nbr,),
                                device_id_type=_DEV_ID_TYPE.MESH)
        pl.semaphore_wait(barrier_sem, 2)

        def signal(sem, dev):
            pl.semaphore_signal(sem, inc=1, device_id=(dev,),
                                device_id_type=_DEV_ID_TYPE.MESH)

        def partial_half(c, half):
            xc = x_ref[pl.ds(c * ch, ch), :]
            wc = w_ref[:, half * n2:(half + 1) * n2]
            return lax.dot_general(
                xc, wc, (((1,), (0,)), ((), ())),
                preferred_element_type=jnp.float32)

        def gelu(y):
            k0 = 0.7978845608028654
            return 0.5 * y * (1.0 + jnp.tanh(k0 * (y + 0.044715 * y * y * y)))

        def store_half(c, half, stage, sem, vals_f32):
            stage[...] = gelu(vals_f32)
            cp = pltpu.make_async_copy(
                stage,
                out_ref.at[pl.ds(c * ch, ch), half * n2:(half + 1) * n2],
                sem)
            cp.start()
            cp.wait()

        def start_hop(s, d):
            pl.semaphore_wait(credit_r, 1)
            pl.semaphore_wait(credit_l, 1)
            rdma_r = pltpu.make_async_remote_copy(
                src_ref=comm_r.at[s], dst_ref=comm_r.at[d],
                send_sem=send_sems_r.at[s], recv_sem=recv_sems_r.at[d],
                device_id=(right,), device_id_type=_DEV_ID_TYPE.MESH)
            rdma_l = pltpu.make_async_remote_copy(
                src_ref=comm_l.at[s], dst_ref=comm_l.at[d],
                send_sem=send_sems_l.at[s], recv_sem=recv_sems_l.at[d],
                device_id=(left,), device_id_type=_DEV_ID_TYPE.MESH)
            rdma_r.start()
            rdma_l.start()
            return rdma_r, rdma_l

        def finish_hop(rdma_r, rdma_l):
            rdma_r.wait()
            rdma_l.wait()
            signal(credit_r, left)
            signal(credit_l, right)

        def rs_hop(h, s, d):
            rdma_r, rdma_l = start_hop(s, d)
            p_r = partial_half(lax.rem(r - h - 1 + N_DEV, N_DEV), 0)
            p_l = partial_half(lax.rem(r + h + 1, N_DEV), 1)
            finish_hop(rdma_r, rdma_l)
            comm_r[d, :, :] = comm_r[d] + p_r.astype(jnp.bfloat16)
            comm_l[d, :, :] = comm_l[d] + p_l.astype(jnp.bfloat16)

        def ag_store(g, d):
            store_half(lax.rem(r - g + N_DEV, N_DEV), 0, stage_r, out_sem_r,
                       comm_r[d].astype(jnp.float32))
            store_half(lax.rem(r + g, N_DEV), 1, stage_l, out_sem_l,
                       comm_l[d].astype(jnp.float32))

        def ag_mid(g, s, d):
            rdma_r = pltpu.make_async_remote_copy(
                src_ref=comm_r.at[s], dst_ref=comm_r.at[d],
                send_sem=send_sems_r.at[s], recv_sem=recv_sems_r.at[d],
                device_id=(right,), device_id_type=_DEV_ID_TYPE.MESH)
            rdma_l = pltpu.make_async_remote_copy(
                src_ref=comm_l.at[s], dst_ref=comm_l.at[d],
                send_sem=send_sems_l.at[s], recv_sem=recv_sems_l.at[d],
                device_id=(left,), device_id_type=_DEV_ID_TYPE.MESH)
            finish_hop(rdma_r, rdma_l)
            start_hop(d, s)
            ag_store(g, d)

        comm_r[0, :, :] = partial_half(r, 0).astype(jnp.bfloat16)
        comm_l[0, :, :] = partial_half(r, 1).astype(jnp.bfloat16)

        signal(credit_r, left)
        signal(credit_l, right)

        def rs_pair(i, carry):
            rs_hop(2 * i, 0, 1)
            rs_hop(2 * i + 1, 1, 0)
            return carry
        lax.fori_loop(0, (N_DEV - 1) // 2, rs_pair, 0)
        rs_hop(N_DEV - 2, 0, 1)

        start_hop(1, 0)
        store_half(lax.rem(r + 1, N_DEV), 0, stage_r, out_sem_r,
                   comm_r[1].astype(jnp.float32))
        store_half(lax.rem(r - 1 + N_DEV, N_DEV), 1, stage_l, out_sem_l,
                   comm_l[1].astype(jnp.float32))

        def ag_pair(j, carry):
            ag_mid(2 * j, 1, 0)
            ag_mid(2 * j + 1, 0, 1)
            return carry
        lax.fori_loop(0, (N_DEV - 1) // 2, ag_pair, 0)

        rdma_r = pltpu.make_async_remote_copy(
            src_ref=comm_r.at[1], dst_ref=comm_r.at[0],
            send_sem=send_sems_r.at[1], recv_sem=recv_sems_r.at[0],
            device_id=(right,), device_id_type=_DEV_ID_TYPE.MESH)
        rdma_l = pltpu.make_async_remote_copy(
            src_ref=comm_l.at[1], dst_ref=comm_l.at[0],
            send_sem=send_sems_l.at[1], recv_sem=recv_sems_l.at[0],
            device_id=(left,), device_id_type=_DEV_ID_TYPE.MESH)
        rdma_r.wait()
        rdma_l.wait()
        ag_store(N_DEV - 2, 0)

    return pl.pallas_call(
        body,
        out_shape=jax.ShapeDtypeStruct((m, n), jnp.float32),
        in_specs=[
            pl.BlockSpec(memory_space=pltpu.VMEM),
            pl.BlockSpec(memory_space=pltpu.VMEM),
        ],
        out_specs=pl.BlockSpec(memory_space=pl.ANY),
        scratch_shapes=[
            pltpu.VMEM((2, ch, n2), jnp.bfloat16),
            pltpu.VMEM((2, ch, n2), jnp.bfloat16),
            pltpu.VMEM((ch, n2), jnp.float32),
            pltpu.VMEM((ch, n2), jnp.float32),
            pltpu.SemaphoreType.DMA((2,)),
            pltpu.SemaphoreType.DMA((2,)),
            pltpu.SemaphoreType.DMA((2,)),
            pltpu.SemaphoreType.DMA((2,)),
            pltpu.SemaphoreType.DMA,
            pltpu.SemaphoreType.DMA,
            pltpu.SemaphoreType.REGULAR,
            pltpu.SemaphoreType.REGULAR,
        ],
        compiler_params=pltpu.CompilerParams(
            collective_id=0, vmem_limit_bytes=100 * 1024 * 1024),
    )(x, w_mat)


# device time: 854727 ns/iter; 1.9260x vs baseline; 1.0152x over previous
import jax
import jax.numpy as jnp
from jax import lax
from jax.experimental import pallas as pl
from jax.experimental.pallas import tpu as pltpu

N_DEV = 16

RING = [0, 4, 8, 12, 13, 9, 5, 1, 2, 6, 10, 14, 15, 11, 7, 3]
RPOS = [0] * N_DEV
NEXT_ID = [0] * N_DEV
PREV_ID = [0] * N_DEV
for _j, _p in enumerate(RING):
    RPOS[_p] = _j
    NEXT_ID[_p] = RING[(_j + 1) % N_DEV]
    PREV_ID[_p] = RING[(_j - 1) % N_DEV]

_DEV_ID_TYPE = getattr(pl, "DeviceIdType", None) or pltpu.DeviceIdType


def _lut(idx, table):
    out = jnp.int32(table[0])
    for k in range(1, len(table)):
        out = jnp.where(idx == k, jnp.int32(table[k]), out)
    return out


def kernel(x, w_mat):
    x = x.astype(jnp.bfloat16)
    w_mat = w_mat.astype(jnp.bfloat16)
    m, k_sh = x.shape
    _, n = w_mat.shape
    ch = m // N_DEV
    n2 = n // 2

    def body(x_ref, w_ref, out_ref, comm_r, comm_l, stage_r, stage_l,
             send_sems_r, recv_sems_r, send_sems_l, recv_sems_l,
             out_sem_r, out_sem_l, credit_r, credit_l):
        my = lax.axis_index("i")
        r = _lut(my, RPOS)
        right = _lut(my, NEXT_ID)
        left = _lut(my, PREV_ID)

        barrier_sem = pltpu.get_barrier_semaphore()
        for nbr in (left, right):
            pl.semaphore_signal(barrier_sem, inc=1, device_id=(nbr,),
                                device_id_type=_DEV_ID_TYPE.MESH)
        pl.semaphore_wait(barrier_sem, 2)

        def signal(sem, dev):
            pl.semaphore_signal(sem, inc=1, device_id=(dev,),
                                device_id_type=_DEV_ID_TYPE.MESH)

        def partial_half(c, half):
            xc = x_ref[pl.ds(c * ch, ch), :]
            wc = w_ref[:, half * n2:(half + 1) * n2]
            return lax.dot_general(
                xc, wc, (((1,), (0,)), ((), ())),
                preferred_element_type=jnp.float32)

        def gelu(y):
            k0 = 0.7978845608028654
            return 0.5 * y * (1.0 + jnp.tanh(k0 * (y + 0.044715 * y * y * y)))

        def store_half(c, half, stage, sem, vals_f32):
            stage[...] = gelu(vals_f32)
            cp = pltpu.make_async_copy(
                stage,
                out_ref.at[pl.ds(c * ch, ch), half * n2:(half + 1) * n2],
                sem)
            cp.start()
            cp.wait()

        def start_hop(s, d):
            pl.semaphore_wait(credit_r, 1)
            pl.semaphore_wait(credit_l, 1)
            rdma_r = pltpu.make_async_remote_copy(
                src_ref=comm_r.at[s], dst_ref=comm_r.at[d],
                send_sem=send_sems_r.at[s], recv_sem=recv_sems_r.at[d],
                device_id=(right,), device_id_type=_DEV_ID_TYPE.MESH)
            rdma_l = pltpu.make_async_remote_copy(
                src_ref=comm_l.at[s], dst_ref=comm_l.at[d],
                send_sem=send_sems_l.at[s], recv_sem=recv_sems_l.at[d],
                device_id=(left,), device_id_type=_DEV_ID_TYPE.MESH)
            rdma_r.start()
            rdma_l.start()
            return rdma_r, rdma_l

        def finish_hop(rdma_r, rdma_l, do_signal=True):
            rdma_r.wait()
            rdma_l.wait()
            if do_signal:
                signal(credit_r, left)
                signal(credit_l, right)

        def rs_hop(h, s, d):
            rdma_r, rdma_l = start_hop(s, d)
            p_r = partial_half(lax.rem(r - h - 1 + N_DEV, N_DEV), 0)
            p_l = partial_half(lax.rem(r + h + 1, N_DEV), 1)
            finish_hop(rdma_r, rdma_l)
            comm_r[d, :, :] = comm_r[d] + p_r.astype(jnp.bfloat16)
            comm_l[d, :, :] = comm_l[d] + p_l.astype(jnp.bfloat16)

        def ag_store(g, d):
            store_half(lax.rem(r - g + N_DEV, N_DEV), 0, stage_r, out_sem_r,
                       comm_r[d].astype(jnp.float32))
            store_half(lax.rem(r + g, N_DEV), 1, stage_l, out_sem_l,
                       comm_l[d].astype(jnp.float32))

        def make_pair(s, d):
            rdma_r = pltpu.make_async_remote_copy(
                src_ref=comm_r.at[s], dst_ref=comm_r.at[d],
                send_sem=send_sems_r.at[s], recv_sem=recv_sems_r.at[d],
                device_id=(right,), device_id_type=_DEV_ID_TYPE.MESH)
            rdma_l = pltpu.make_async_remote_copy(
                src_ref=comm_l.at[s], dst_ref=comm_l.at[d],
                send_sem=send_sems_l.at[s], recv_sem=recv_sems_l.at[d],
                device_id=(left,), device_id_type=_DEV_ID_TYPE.MESH)
            return rdma_r, rdma_l

        def ag_mid(g, s, d, e, do_signal=True):
            rdma_r, rdma_l = make_pair(s, d)
            finish_hop(rdma_r, rdma_l, do_signal)
            start_hop(d, e)
            ag_store(g, d)

        comm_r[0, :, :] = partial_half(r, 0).astype(jnp.bfloat16)
        comm_l[0, :, :] = partial_half(r, 1).astype(jnp.bfloat16)

        for _ in range(2):
            signal(credit_r, left)
            signal(credit_l, right)

        def rs_trip(i, carry):
            rs_hop(3 * i, 0, 1)
            rs_hop(3 * i + 1, 1, 2)
            rs_hop(3 * i + 2, 2, 0)
            return carry
        lax.fori_loop(0, (N_DEV - 1) // 3, rs_trip, 0)

        start_hop(0, 1)
        store_half(lax.rem(r + 1, N_DEV), 0, stage_r, out_sem_r,
                   comm_r[0].astype(jnp.float32))
        store_half(lax.rem(r - 1 + N_DEV, N_DEV), 1, stage_l, out_sem_l,
                   comm_l[0].astype(jnp.float32))

        def ag_trip(j, carry):
            ag_mid(3 * j, 0, 1, 2)
            ag_mid(3 * j + 1, 1, 2, 0)
            ag_mid(3 * j + 2, 2, 0, 1)
            return carry
        lax.fori_loop(0, 4, ag_trip, 0)
        ag_mid(12, 0, 1, 2)
        ag_mid(13, 1, 2, 0, do_signal=False)

        rdma_r, rdma_l = make_pair(2, 0)
        rdma_r.wait()
        rdma_l.wait()
        ag_store(N_DEV - 2, 0)

    return pl.pallas_call(
        body,
        out_shape=jax.ShapeDtypeStruct((m, n), jnp.float32),
        in_specs=[
            pl.BlockSpec(memory_space=pltpu.VMEM),
            pl.BlockSpec(memory_space=pltpu.VMEM),
        ],
        out_specs=pl.BlockSpec(memory_space=pl.ANY),
        scratch_shapes=[
            pltpu.VMEM((3, ch, n2), jnp.bfloat16),
            pltpu.VMEM((3, ch, n2), jnp.bfloat16),
            pltpu.VMEM((ch, n2), jnp.float32),
            pltpu.VMEM((ch, n2), jnp.float32),
            pltpu.SemaphoreType.DMA((3,)),
            pltpu.SemaphoreType.DMA((3,)),
            pltpu.SemaphoreType.DMA((3,)),
            pltpu.SemaphoreType.DMA((3,)),
            pltpu.SemaphoreType.DMA,
            pltpu.SemaphoreType.DMA,
            pltpu.SemaphoreType.REGULAR,
            pltpu.SemaphoreType.REGULAR,
        ],
        compiler_params=pltpu.CompilerParams(
            collective_id=0, vmem_limit_bytes=100 * 1024 * 1024),
    )(x, w_mat)


# device time: 796513 ns/iter; 2.0668x vs baseline; 1.0731x over previous
import jax
import jax.numpy as jnp
from jax import lax
from jax.experimental import pallas as pl
from jax.experimental.pallas import tpu as pltpu

N_DEV = 16

RING = [0, 4, 8, 12, 13, 9, 5, 1, 2, 6, 10, 14, 15, 11, 7, 3]
RPOS = [0] * N_DEV
NEXT_ID = [0] * N_DEV
PREV_ID = [0] * N_DEV
for _j, _p in enumerate(RING):
    RPOS[_p] = _j
    NEXT_ID[_p] = RING[(_j + 1) % N_DEV]
    PREV_ID[_p] = RING[(_j - 1) % N_DEV]

_DEV_ID_TYPE = getattr(pl, "DeviceIdType", None) or pltpu.DeviceIdType


def _lut(idx, table):
    out = jnp.int32(table[0])
    for k in range(1, len(table)):
        out = jnp.where(idx == k, jnp.int32(table[k]), out)
    return out


def kernel(x, w_mat):
    x = x.astype(jnp.bfloat16)
    w_mat = w_mat.astype(jnp.bfloat16)
    m, k_sh = x.shape
    _, n = w_mat.shape
    ch = m // N_DEV
    h2 = ch // 2
    n2 = n // 2

    def body(x_ref, w_ref, out_ref,
             comm_ar, comm_br, comm_al, comm_bl, stage_r, stage_l,
             send_ar, recv_ar, send_br, recv_br,
             send_al, recv_al, send_bl, recv_bl,
             out_sem_r, out_sem_l,
             credit_ar, credit_br, credit_al, credit_bl):
        my = lax.axis_index("i")
        r = _lut(my, RPOS)
        right = _lut(my, NEXT_ID)
        left = _lut(my, PREV_ID)

        barrier_sem = pltpu.get_barrier_semaphore()
        for nbr in (left, right):
            pl.semaphore_signal(barrier_sem, inc=1, device_id=(nbr,),
                                device_id_type=_DEV_ID_TYPE.MESH)
        pl.semaphore_wait(barrier_sem, 2)

        def signal(sem, dev):
            pl.semaphore_signal(sem, inc=1, device_id=(dev,),
                                device_id_type=_DEV_ID_TYPE.MESH)

        def partial_q(c, half, sub):
            xc = x_ref[pl.ds(c * ch + sub * h2, h2), :]
            wc = w_ref[:, half * n2:(half + 1) * n2]
            return lax.dot_general(
                xc, wc, (((1,), (0,)), ((), ())),
                preferred_element_type=jnp.float32)

        def gelu(y):
            k0 = 0.7978845608028654
            return 0.5 * y * (1.0 + jnp.tanh(k0 * (y + 0.044715 * y * y * y)))

        def mk(buf, s, d, sems_s, sems_r, dev):
            return pltpu.make_async_remote_copy(
                src_ref=buf.at[s], dst_ref=buf.at[d],
                send_sem=sems_s.at[s], recv_sem=sems_r.at[d],
                device_id=(dev,), device_id_type=_DEV_ID_TYPE.MESH)

        def mk_a(s, d):
            return (mk(comm_ar, s, d, send_ar, recv_ar, right),
                    mk(comm_al, s, d, send_al, recv_al, left))

        def mk_b(s, d):
            return (mk(comm_br, s, d, send_br, recv_br, right),
                    mk(comm_bl, s, d, send_bl, recv_bl, left))

        def start_a(s, d):
            pl.semaphore_wait(credit_ar, 1)
            pl.semaphore_wait(credit_al, 1)
            a_r, a_l = mk_a(s, d)
            a_r.start()
            a_l.start()

        def start_b(s, d):
            pl.semaphore_wait(credit_br, 1)
            pl.semaphore_wait(credit_bl, 1)
            b_r, b_l = mk_b(s, d)
            b_r.start()
            b_l.start()

        def wait_a(s, d, do_signal=True):
            a_r, a_l = mk_a(s, d)
            a_r.wait()
            a_l.wait()
            if do_signal:
                signal(credit_ar, left)
                signal(credit_al, right)

        def wait_b(s, d, do_signal=True):
            b_r, b_l = mk_b(s, d)
            b_r.wait()
            b_l.wait()
            if do_signal:
                signal(credit_br, left)
                signal(credit_bl, right)

        def rs_mid(h, s, d, e, start_next=True):
            c_r = lax.rem(r - h - 1 + N_DEV, N_DEV)
            c_l = lax.rem(r + h + 1, N_DEV)
            pa_r = partial_q(c_r, 0, 0).astype(jnp.bfloat16)
            pa_l = partial_q(c_l, 1, 0).astype(jnp.bfloat16)
            pb_r = partial_q(c_r, 0, 1).astype(jnp.bfloat16)
            pb_l = partial_q(c_l, 1, 1).astype(jnp.bfloat16)
            wait_a(s, d)
            comm_ar[d, :, :] = comm_ar[d] + pa_r
            comm_al[d, :, :] = comm_al[d] + pa_l
            if start_next:
                start_a(d, e)
            wait_b(s, d)
            comm_br[d, :, :] = comm_br[d] + pb_r
            comm_bl[d, :, :] = comm_bl[d] + pb_l
            if start_next:
                start_b(d, e)

        def ag_store(g, d):
            c_r = lax.rem(r - g + N_DEV, N_DEV)
            c_l = lax.rem(r + g, N_DEV)
            stage_r[0:h2, :] = gelu(comm_ar[d].astype(jnp.float32))
            stage_r[h2:ch, :] = gelu(comm_br[d].astype(jnp.float32))
            stage_l[0:h2, :] = gelu(comm_al[d].astype(jnp.float32))
            stage_l[h2:ch, :] = gelu(comm_bl[d].astype(jnp.float32))
            cp_r = pltpu.make_async_copy(
                stage_r, out_ref.at[pl.ds(c_r * ch, ch), 0:n2], out_sem_r)
            cp_l = pltpu.make_async_copy(
                stage_l, out_ref.at[pl.ds(c_l * ch, ch), n2:n], out_sem_l)
            cp_r.start()
            cp_l.start()
            cp_r.wait()
            cp_l.wait()

        def ag_mid(g, s, d, e, do_signal=True):
            wait_a(s, d, do_signal)
            start_a(d, e)
            wait_b(s, d, do_signal)
            start_b(d, e)
            ag_store(g, d)

        comm_ar[0, :, :] = partial_q(r, 0, 0).astype(jnp.bfloat16)
        comm_al[0, :, :] = partial_q(r, 1, 0).astype(jnp.bfloat16)
        comm_br[0, :, :] = partial_q(r, 0, 1).astype(jnp.bfloat16)
        comm_bl[0, :, :] = partial_q(r, 1, 1).astype(jnp.bfloat16)

        for _ in range(2):
            signal(credit_ar, left)
            signal(credit_al, right)
            signal(credit_br, left)
            signal(credit_bl, right)

        start_a(0, 1)
        start_b(0, 1)

        def rs_trip(i, carry):
            h = 3 * i
            rs_mid(h, 0, 1, 2)
            rs_mid(h + 1, 1, 2, 0)
            rs_mid(h + 2, 2, 0, 1)
            return carry
        lax.fori_loop(0, 4, rs_trip, 0)
        rs_mid(12, 0, 1, 2)
        rs_mid(13, 1, 2, 0)
        rs_mid(14, 2, 0, 1)

        stage_r[0:h2, :] = gelu(comm_ar[0].astype(jnp.float32))
        stage_r[h2:ch, :] = gelu(comm_br[0].astype(jnp.float32))
        stage_l[0:h2, :] = gelu(comm_al[0].astype(jnp.float32))
        stage_l[h2:ch, :] = gelu(comm_bl[0].astype(jnp.float32))
        c_own_r = lax.rem(r + 1, N_DEV)
        c_own_l = lax.rem(r - 1 + N_DEV, N_DEV)
        cp_r = pltpu.make_async_copy(
            stage_r, out_ref.at[pl.ds(c_own_r * ch, ch), 0:n2], out_sem_r)
        cp_l = pltpu.make_async_copy(
            stage_l, out_ref.at[pl.ds(c_own_l * ch, ch), n2:n], out_sem_l)
        cp_r.start()
        cp_l.start()
        cp_r.wait()
        cp_l.wait()

        def ag_trip(j, carry):
            g = 3 * j
            ag_mid(g, 0, 1, 2)
            ag_mid(g + 1, 1, 2, 0)
            ag_mid(g + 2, 2, 0, 1)
            return carry
        lax.fori_loop(0, 4, ag_trip, 0)
        ag_mid(12, 0, 1, 2)
        ag_mid(13, 1, 2, 0, do_signal=False)

        wait_a(2, 0, do_signal=False)
        wait_b(2, 0, do_signal=False)
        ag_store(N_DEV - 2, 0)

    return pl.pallas_call(
        body,
        out_shape=jax.ShapeDtypeStruct((m, n), jnp.float32),
        in_specs=[
            pl.BlockSpec(memory_space=pltpu.VMEM),
            pl.BlockSpec(memory_space=pltpu.VMEM),
        ],
        out_specs=pl.BlockSpec(memory_space=pl.ANY),
        scratch_shapes=[
            pltpu.VMEM((3, h2, n2), jnp.bfloat16),
            pltpu.VMEM((3, h2, n2), jnp.bfloat16),
            pltpu.VMEM((3, h2, n2), jnp.bfloat16),
            pltpu.VMEM((3, h2, n2), jnp.bfloat16),
            pltpu.VMEM((ch, n2), jnp.float32),
            pltpu.VMEM((ch, n2), jnp.float32),
            pltpu.SemaphoreType.DMA((3,)),
            pltpu.SemaphoreType.DMA((3,)),
            pltpu.SemaphoreType.DMA((3,)),
            pltpu.SemaphoreType.DMA((3,)),
            pltpu.SemaphoreType.DMA((3,)),
            pltpu.SemaphoreType.DMA((3,)),
            pltpu.SemaphoreType.DMA((3,)),
            pltpu.SemaphoreType.DMA((3,)),
            pltpu.SemaphoreType.DMA,
            pltpu.SemaphoreType.DMA,
            pltpu.SemaphoreType.REGULAR,
            pltpu.SemaphoreType.REGULAR,
            pltpu.SemaphoreType.REGULAR,
            pltpu.SemaphoreType.REGULAR,
        ],
        compiler_params=pltpu.CompilerParams(
            collective_id=0, vmem_limit_bytes=100 * 1024 * 1024),
    )(x, w_mat)


# device time: 795854 ns/iter; 2.0685x vs baseline; 1.0008x over previous
import jax
import jax.numpy as jnp
from jax import lax
from jax.experimental import pallas as pl
from jax.experimental.pallas import tpu as pltpu

N_DEV = 16
SUBS = 4

RING = [0, 4, 8, 12, 13, 9, 5, 1, 2, 6, 10, 14, 15, 11, 7, 3]
RPOS = [0] * N_DEV
NEXT_ID = [0] * N_DEV
PREV_ID = [0] * N_DEV
for _j, _p in enumerate(RING):
    RPOS[_p] = _j
    NEXT_ID[_p] = RING[(_j + 1) % N_DEV]
    PREV_ID[_p] = RING[(_j - 1) % N_DEV]

_DEV_ID_TYPE = getattr(pl, "DeviceIdType", None) or pltpu.DeviceIdType


def _lut(idx, table):
    out = jnp.int32(table[0])
    for k in range(1, len(table)):
        out = jnp.where(idx == k, jnp.int32(table[k]), out)
    return out


def kernel(x, w_mat):
    x = x.astype(jnp.bfloat16)
    w_mat = w_mat.astype(jnp.bfloat16)
    m, k_sh = x.shape
    _, n = w_mat.shape
    ch = m // N_DEV
    q = ch // SUBS
    n2 = n // 2

    def body(x_ref, w_ref, out_ref, *scr):
        comm = [[scr[ring * SUBS + k] for k in range(SUBS)]
                for ring in range(2)]
        stage = [scr[2 * SUBS], scr[2 * SUBS + 1]]
        base = 2 * SUBS + 2
        ssem = [[scr[base + (ring * SUBS + k) * 2] for k in range(SUBS)]
                for ring in range(2)]
        rsem = [[scr[base + (ring * SUBS + k) * 2 + 1] for k in range(SUBS)]
                for ring in range(2)]
        base += 4 * SUBS
        out_sem = [scr[base], scr[base + 1]]
        base += 2
        credit = [[scr[base + ring * SUBS + k] for k in range(SUBS)]
                  for ring in range(2)]

        my = lax.axis_index("i")
        r = _lut(my, RPOS)
        right = _lut(my, NEXT_ID)
        left = _lut(my, PREV_ID)
        send_to = [right, left]
        recv_from = [left, right]

        barrier_sem = pltpu.get_barrier_semaphore()
        for nbr in (left, right):
            pl.semaphore_signal(barrier_sem, inc=1, device_id=(nbr,),
                                device_id_type=_DEV_ID_TYPE.MESH)
        pl.semaphore_wait(barrier_sem, 2)

        def signal(sem, dev):
            pl.semaphore_signal(sem, inc=1, device_id=(dev,),
                                device_id_type=_DEV_ID_TYPE.MESH)

        def partial_half(c, ring):
            xc = x_ref[pl.ds(c * ch, ch), :]
            wc = w_ref[:, ring * n2:(ring + 1) * n2]
            return lax.dot_general(
                xc, wc, (((1,), (0,)), ((), ())),
                preferred_element_type=jnp.float32).astype(jnp.bfloat16)

        def gelu(y):
            k0 = 0.7978845608028654
            return 0.5 * y * (1.0 + jnp.tanh(k0 * (y + 0.044715 * y * y * y)))

        def mk(ring, k, s, d):
            return pltpu.make_async_remote_copy(
                src_ref=comm[ring][k].at[s], dst_ref=comm[ring][k].at[d],
                send_sem=ssem[ring][k].at[s], recv_sem=rsem[ring][k].at[d],
                device_id=(send_to[ring],),
                device_id_type=_DEV_ID_TYPE.MESH)

        def start_sub(k, s, d):
            for ring in range(2):
                pl.semaphore_wait(credit[ring][k], 1)
            for ring in range(2):
                mk(ring, k, s, d).start()

        def wait_sub(k, s, d, do_signal=True):
            for ring in range(2):
                mk(ring, k, s, d).wait()
            if do_signal:
                for ring in range(2):
                    signal(credit[ring][k], recv_from[ring])

        def rs_mid(h, s, d, e, start_next=True):
            c = [lax.rem(r - h - 1 + N_DEV, N_DEV),
                 lax.rem(r + h + 1, N_DEV)]
            p = [partial_half(c[0], 0), partial_half(c[1], 1)]
            for k in range(SUBS):
                wait_sub(k, s, d)
                for ring in range(2):
                    comm[ring][k][d, :, :] = (
                        comm[ring][k][d] + p[ring][k * q:(k + 1) * q, :])
                if start_next:
                    start_sub(k, d, e)

        def ag_store(g, d):
            c = [lax.rem(r - g + N_DEV, N_DEV), lax.rem(r + g, N_DEV)]
            for ring in range(2):
                for k in range(SUBS):
                    stage[ring][k * q:(k + 1) * q, :] = gelu(
                        comm[ring][k][d].astype(jnp.float32))
            cps = []
            for ring in range(2):
                cp = pltpu.make_async_copy(
                    stage[ring],
                    out_ref.at[pl.ds(c[ring] * ch, ch),
                               ring * n2:(ring + 1) * n2],
                    out_sem[ring])
                cp.start()
                cps.append(cp)
            for cp in cps:
                cp.wait()

        def ag_mid(g, s, d, e, do_signal=True):
            for k in range(SUBS):
                wait_sub(k, s, d, do_signal)
                start_sub(k, d, e)
            ag_store(g, d)

        p0 = [partial_half(r, 0), partial_half(r, 1)]
        for ring in range(2):
            for k in range(SUBS):
                comm[ring][k][0, :, :] = p0[ring][k * q:(k + 1) * q, :]

        for _ in range(2):
            for ring in range(2):
                for k in range(SUBS):
                    signal(credit[ring][k], recv_from[ring])

        for k in range(SUBS):
            start_sub(k, 0, 1)

        def rs_trip(i, carry):
            h = 3 * i
            rs_mid(h, 0, 1, 2)
            rs_mid(h + 1, 1, 2, 0)
            rs_mid(h + 2, 2, 0, 1)
            return carry
        lax.fori_loop(0, 4, rs_trip, 0)
        rs_mid(12, 0, 1, 2)
        rs_mid(13, 1, 2, 0)
        rs_mid(14, 2, 0, 1)

        c_own = [lax.rem(r + 1, N_DEV), lax.rem(r - 1 + N_DEV, N_DEV)]
        for ring in range(2):
            for k in range(SUBS):
                stage[ring][k * q:(k + 1) * q, :] = gelu(
                    comm[ring][k][0].astype(jnp.float32))
        cps = []
        for ring in range(2):
            cp = pltpu.make_async_copy(
                stage[ring],
                out_ref.at[pl.ds(c_own[ring] * ch, ch),
                           ring * n2:(ring + 1) * n2],
                out_sem[ring])
            cp.start()
            cps.append(cp)
        for cp in cps:
            cp.wait()

        def ag_trip(j, carry):
            g = 3 * j
            ag_mid(g, 0, 1, 2)
            ag_mid(g + 1, 1, 2, 0)
            ag_mid(g + 2, 2, 0, 1)
            return carry
        lax.fori_loop(0, 4, ag_trip, 0)
        ag_mid(12, 0, 1, 2)
        ag_mid(13, 1, 2, 0, do_signal=False)

        for k in range(SUBS):
            wait_sub(k, 2, 0, do_signal=False)
        ag_store(N_DEV - 2, 0)

    scratch = (
        [pltpu.VMEM((3, q, n2), jnp.bfloat16) for _ in range(2 * SUBS)]
        + [pltpu.VMEM((ch, n2), jnp.float32) for _ in range(2)]
        + [pltpu.SemaphoreType.DMA((3,)) for _ in range(4 * SUBS)]
        + [pltpu.SemaphoreType.DMA for _ in range(2)]
        + [pltpu.SemaphoreType.REGULAR for _ in range(2 * SUBS)]
    )

    return pl.pallas_call(
        body,
        out_shape=jax.ShapeDtypeStruct((m, n), jnp.float32),
        in_specs=[
            pl.BlockSpec(memory_space=pltpu.VMEM),
            pl.BlockSpec(memory_space=pltpu.VMEM),
        ],
        out_specs=pl.BlockSpec(memory_space=pl.ANY),
        scratch_shapes=scratch,
        compiler_params=pltpu.CompilerParams(
            collective_id=0, vmem_limit_bytes=100 * 1024 * 1024),
    )(x, w_mat)
